# baseline (speedup 1.0000x reference)
import functools

import jax
import jax.numpy as jnp
from jax import lax
from jax.experimental import pallas as pl
from jax.experimental.pallas import tpu as pltpu
from jax.experimental.pallas import tpu_sc as plsc

NC = 2
NS = 16
NW = NC * NS
CHUNK = 128
SUPER = 2
NBUF = 2


def _emb_lookup(idx, table, total, D, V):
    per_w = total // NW
    nchunks_w = per_w // CHUNK
    nsuper = nchunks_w // SUPER
    mesh = plsc.VectorSubcoreMesh(core_axis_name="c", subcore_axis_name="s")

    @functools.partial(
        pl.kernel,
        out_type=jax.ShapeDtypeStruct((total, D), jnp.float32),
        mesh=mesh,
        scratch_types=[
            pltpu.VMEM((nchunks_w, CHUNK), jnp.int32),
            pltpu.VMEM((NBUF, SUPER * CHUNK, D), jnp.float32),
            pltpu.VMEM_SHARED((V, D), jnp.float32),
        ]
        + [pltpu.SemaphoreType.DMA] * (2 * NBUF),
    )
    def body(idx_hbm, table_hbm, out_hbm, idx_v, bufs, table_sp, *sems):
        gsem = sems[:NBUF]
        wsem = sems[NBUF:]
        sid = lax.axis_index("s")
        wid = lax.axis_index("c") * NS + sid
        base = wid * per_w

        @pl.when(sid == 0)
        def _():
            pltpu.sync_copy(table_hbm, table_sp)

        pltpu.sync_copy(idx_hbm.at[wid], idx_v)
        plsc.subcore_barrier()

        def start_gathers(s, b):
            for c in range(SUPER):
                pltpu.async_copy(
                    table_sp.at[idx_v.at[s * SUPER + c]],
                    bufs.at[b].at[pl.ds(c * CHUNK, CHUNK)],
                    gsem[b],
                )

        def wait_gathers(b):
            pltpu.make_async_copy(
                table_sp.at[pl.ds(0, SUPER * CHUNK)], bufs.at[b], gsem[b]
            ).wait()

        def start_write(s, b):
            pltpu.async_copy(
                bufs.at[b],
                out_hbm.at[pl.ds(base + s * SUPER * CHUNK, SUPER * CHUNK)],
                wsem[b],
            )

        def wait_write(b):
            pltpu.make_async_copy(
                bufs.at[b], out_hbm.at[pl.ds(base, SUPER * CHUNK)], wsem[b]
            ).wait()

        for b in range(NBUF):
            start_gathers(b, b)

        ngroups = nsuper // NBUF

        @pl.loop(0, ngroups - 1)
        def _(g):
            s0 = g * NBUF
            for b in range(NBUF):
                wait_gathers(b)
                start_write(s0 + b, b)
            for b in range(NBUF):
                wait_write(b)
                start_gathers(s0 + NBUF + b, b)

        s0 = (ngroups - 1) * NBUF
        for b in range(NBUF):
            wait_gathers(b)
            start_write(s0 + b, b)
        for b in range(NBUF):
            wait_write(b)

    return body(idx, table)


def kernel(time, emb_weight):
    B, H = time.shape
    V, D = emb_weight.shape
    total = B * H
    idx = time.reshape(NW, (total // NW) // CHUNK, CHUNK).astype(jnp.int32)
    out = _emb_lookup(idx, emb_weight, total, D, V)
    return out.reshape(B, H, D)


# speedup vs baseline: 1.0227x; 1.0227x over previous
import functools

import jax
import jax.numpy as jnp
from jax import lax
from jax.experimental import pallas as pl
from jax.experimental.pallas import tpu as pltpu
from jax.experimental.pallas import tpu_sc as plsc

NC = 2
NS = 16
NW = NC * NS
CHUNK = 128
NBUF = 4


def _emb_lookup(idx, table, total, D, V):
    per_w = total // NW
    nchunks_w = per_w // CHUNK
    mesh = plsc.VectorSubcoreMesh(core_axis_name="c", subcore_axis_name="s")

    @functools.partial(
        pl.kernel,
        out_type=jax.ShapeDtypeStruct((total, D), jnp.float32),
        mesh=mesh,
        scratch_types=[
            pltpu.VMEM((nchunks_w, CHUNK), jnp.int32),
            pltpu.VMEM((NBUF, CHUNK, D), jnp.float32),
            pltpu.VMEM_SHARED((V, D), jnp.float32),
            pltpu.VMEM_SHARED((NS, 1, CHUNK, D), jnp.float32),
        ]
        + [pltpu.SemaphoreType.DMA] * (NBUF + 5),
    )
    def body(idx_hbm, table_hbm, out_hbm, idx_v, bufs, table_sp, spb_all, *sems):
        gsem = sems[:NBUF]
        wsem = sems[NBUF:NBUF + 3]
        csem = sems[NBUF + 3:NBUF + 4]
        dsem = sems[NBUF + 4:NBUF + 5]
        sid = lax.axis_index("s")
        wid = lax.axis_index("c") * NS + sid
        base = wid * per_w
        spb = spb_all.at[sid]

        @pl.when(sid == 0)
        def _():
            pltpu.sync_copy(table_hbm, table_sp)

        pltpu.sync_copy(idx_hbm.at[wid], idx_v)
        plsc.subcore_barrier()

        def start_gather(j, b):
            pltpu.async_copy(table_sp.at[idx_v.at[j]], bufs.at[b], gsem[b])

        def wait_gather(b):
            pltpu.make_async_copy(
                table_sp.at[pl.ds(0, CHUNK)], bufs.at[b], gsem[b]
            ).wait()

        def start_write(j, b):
            pltpu.async_copy(
                bufs.at[b], out_hbm.at[pl.ds(base + j * CHUNK, CHUNK)], wsem[b]
            )

        def wait_write(b):
            pltpu.make_async_copy(
                bufs.at[b], out_hbm.at[pl.ds(base, CHUNK)], wsem[b]
            ).wait()

        def start_copy(k):
            pltpu.async_copy(bufs.at[3 + k], spb.at[k], csem[k])

        def wait_copy(k):
            pltpu.make_async_copy(bufs.at[3 + k], spb.at[k], csem[k]).wait()

        def start_dma(j, k):
            pltpu.async_copy(
                spb.at[k], out_hbm.at[pl.ds(base + j * CHUNK, CHUNK)], dsem[k]
            )

        def wait_dma(k):
            pltpu.make_async_copy(
                spb.at[k], out_hbm.at[pl.ds(base, CHUNK)], dsem[k]
            ).wait()

        for b in range(NBUF):
            start_gather(b, b)

        ngroups = nchunks_w // NBUF

        @pl.loop(0, ngroups - 1)
        def _(g):
            j0 = g * NBUF
            for b in range(3):
                wait_gather(b)
                start_write(j0 + b, b)
            for k in range(1):
                wait_gather(3 + k)

                @pl.when(g > 0)
                def _():
                    wait_dma(k)

                start_copy(k)
            for b in range(3):
                wait_write(b)
                start_gather(j0 + NBUF + b, b)
            for k in range(1):
                wait_copy(k)
                start_dma(j0 + 3 + k, k)
                start_gather(j0 + NBUF + 3 + k, 3 + k)

        j0 = (ngroups - 1) * NBUF
        for b in range(3):
            wait_gather(b)
            start_write(j0 + b, b)
        for k in range(1):
            wait_gather(3 + k)
            wait_dma(k)
            start_copy(k)
        for b in range(3):
            wait_write(b)
        for k in range(1):
            wait_copy(k)
            start_dma(j0 + 3 + k, k)
        for k in range(1):
            wait_dma(k)

    return body(idx, table)


def kernel(time, emb_weight):
    B, H = time.shape
    V, D = emb_weight.shape
    total = B * H
    idx = time.reshape(NW, (total // NW) // CHUNK, CHUNK).astype(jnp.int32)
    out = _emb_lookup(idx, emb_weight, total, D, V)
    return out.reshape(B, H, D)


# speedup vs baseline: 1.4552x; 1.4229x over previous
import functools

import jax
import jax.numpy as jnp
from jax import lax
from jax.experimental import pallas as pl
from jax.experimental.pallas import tpu as pltpu
from jax.experimental.pallas import tpu_sc as plsc

NC = 2
NS = 16
NW = NC * NS
CHUNK = 128
NBUF = 4


def _emb_lookup(idx, table, total, D, V):
    per_w = total // NW
    nchunks_w = per_w // CHUNK
    mesh = plsc.VectorSubcoreMesh(core_axis_name="c", subcore_axis_name="s")

    @functools.partial(
        pl.kernel,
        out_type=jax.ShapeDtypeStruct((total, D), jnp.float32),
        mesh=mesh,
        scratch_types=[
            pltpu.VMEM((nchunks_w, CHUNK), jnp.int32),
            pltpu.VMEM((NBUF, CHUNK, D), jnp.float32),
            pltpu.VMEM_SHARED((V, D), jnp.float32),
        ]
        + [pltpu.SemaphoreType.DMA] * (2 * NBUF),
    )
    def body(idx_hbm, table_hbm, out_hbm, idx_v, bufs, table_sp, *sems):
        gsem = sems[:NBUF]
        wsem = sems[NBUF:]
        sid = lax.axis_index("s")
        wid = lax.axis_index("c") * NS + sid
        base = wid * per_w

        @pl.when(sid == 0)
        def _():
            pltpu.sync_copy(table_hbm, table_sp)

        pltpu.sync_copy(idx_hbm.at[wid], idx_v)
        plsc.subcore_barrier()

        def start_gather(j, b):
            pltpu.async_copy(table_sp.at[idx_v.at[j]], bufs.at[b], gsem[b])

        def wait_gather(b):
            pltpu.make_async_copy(
                table_sp.at[pl.ds(0, CHUNK)], bufs.at[b], gsem[b]
            ).wait()

        def start_write(j, b):
            pltpu.async_copy(
                bufs.at[b], out_hbm.at[pl.ds(base + j * CHUNK, CHUNK)], wsem[b]
            )

        def wait_write(b):
            pltpu.make_async_copy(
                bufs.at[b], out_hbm.at[pl.ds(base, CHUNK)], wsem[b]
            ).wait()

        for b in range(NBUF):
            start_gather(b, b)

        ngroups = nchunks_w // NBUF

        @pl.loop(0, ngroups - 1)
        def _(g):
            j0 = g * NBUF
            for b in range(NBUF):
                wait_gather(b)
                start_write(j0 + b, b)
            for b in range(NBUF):
                wait_write(b)
                start_gather(j0 + NBUF + b, b)

        j0 = (ngroups - 1) * NBUF
        for b in range(NBUF):
            wait_gather(b)
            start_write(j0 + b, b)
        for b in range(NBUF):
            wait_write(b)

    return body(idx, table)


def kernel(time, emb_weight):
    B, H = time.shape
    V, D = emb_weight.shape
    total = B * H
    idx = time.reshape(NW, (total // NW) // CHUNK, CHUNK).astype(jnp.int32)
    out = _emb_lookup(idx, emb_weight, total, D, V)
    return out.reshape(B, H, D)


# speedup vs baseline: 1.4608x; 1.0038x over previous
import functools

import jax
import jax.numpy as jnp
from jax import lax
from jax.experimental import pallas as pl
from jax.experimental.pallas import tpu as pltpu
from jax.experimental.pallas import tpu_sc as plsc

NC = 2
NS = 16
NW = NC * NS
CHUNK = 64
NBUF = 8


def _emb_lookup(idx, table, total, D, V):
    per_w = total // NW
    nchunks_w = per_w // CHUNK
    mesh = plsc.VectorSubcoreMesh(core_axis_name="c", subcore_axis_name="s")

    @functools.partial(
        pl.kernel,
        out_type=jax.ShapeDtypeStruct((total, D), jnp.float32),
        mesh=mesh,
        scratch_types=[
            pltpu.VMEM((nchunks_w, CHUNK), jnp.int32),
            pltpu.VMEM((NBUF, CHUNK, D), jnp.float32),
            pltpu.VMEM_SHARED((V, D), jnp.float32),
        ]
        + [pltpu.SemaphoreType.DMA] * (2 * NBUF),
    )
    def body(idx_hbm, table_hbm, out_hbm, idx_v, bufs, table_sp, *sems):
        gsem = sems[:NBUF]
        wsem = sems[NBUF:]
        sid = lax.axis_index("s")
        wid = lax.axis_index("c") * NS + sid
        base = wid * per_w

        @pl.when(sid == 0)
        def _():
            pltpu.sync_copy(table_hbm, table_sp)

        pltpu.sync_copy(idx_hbm.at[wid], idx_v)
        plsc.subcore_barrier()

        def start_gather(j, b):
            pltpu.async_copy(table_sp.at[idx_v.at[j]], bufs.at[b], gsem[b])

        def wait_gather(b):
            pltpu.make_async_copy(
                table_sp.at[pl.ds(0, CHUNK)], bufs.at[b], gsem[b]
            ).wait()

        def start_write(j, b):
            pltpu.async_copy(
                bufs.at[b], out_hbm.at[pl.ds(base + j * CHUNK, CHUNK)], wsem[b]
            )

        def wait_write(b):
            pltpu.make_async_copy(
                bufs.at[b], out_hbm.at[pl.ds(base, CHUNK)], wsem[b]
            ).wait()

        for b in range(NBUF):
            start_gather(b, b)

        ngroups = nchunks_w // NBUF

        @pl.loop(0, ngroups - 1)
        def _(g):
            j0 = g * NBUF
            for b in range(NBUF):
                wait_gather(b)
                start_write(j0 + b, b)
            for b in range(NBUF):
                wait_write(b)
                start_gather(j0 + NBUF + b, b)

        j0 = (ngroups - 1) * NBUF
        for b in range(NBUF):
            wait_gather(b)
            start_write(j0 + b, b)
        for b in range(NBUF):
            wait_write(b)

    return body(idx, table)


def kernel(time, emb_weight):
    B, H = time.shape
    V, D = emb_weight.shape
    total = B * H
    idx = time.reshape(NW, (total // NW) // CHUNK, CHUNK).astype(jnp.int32)
    out = _emb_lookup(idx, emb_weight, total, D, V)
    return out.reshape(B, H, D)
